# baseline (speedup 1.0000x reference)
import functools

import jax
import jax.numpy as jnp
from jax import lax
from jax.experimental import pallas as pl
from jax.experimental.pallas import tpu as pltpu
from jax.experimental.pallas import tpu_sc as plsc

N = 10000
NPAD = 10240
E = 320000
D = 128
NC = 2
NS = 16
NW = NC * NS
EPW = E // NW
CHUNK = 128
MCH = 64
NCH = 80
EPW_PAD = NCH * CHUNK
ROWS_PER_TILE = NPAD // NS

@functools.cache
def _mesh():
    return plsc.VectorSubcoreMesh(core_axis_name="c", subcore_axis_name="s",
                                  num_cores=NC, num_subcores=NS)


def _zero_fill(buf, rows, cols, val=0.0):
    v = jnp.full((16,), val, jnp.float32)

    def row(r, _):
        def col(c, __):
            buf[r, pl.ds(c * 16, 16)] = v
            return __
        return lax.fori_loop(0, cols // 16, col, _)

    lax.fori_loop(0, rows, row, None)


def _deg_body(dstp_hbm, out_hbm, dst_v, ones_v, sem, acc):
    cid = lax.axis_index("c")
    sid = lax.axis_index("s")
    wid = sid * NC + cid
    pltpu.sync_copy(dstp_hbm.at[wid], dst_v)
    _zero_fill(ones_v, CHUNK, D, 0.0)
    base = sid * ROWS_PER_TILE
    for i in range(ROWS_PER_TILE // CHUNK):
        pltpu.sync_copy(ones_v, acc.at[pl.ds(base + i * CHUNK, CHUNK)])
    _zero_fill(ones_v, CHUNK, D, 1.0)
    plsc.subcore_barrier()

    def body(j, _):
        pltpu.sync_copy(ones_v, acc.at[dst_v.at[j]], add=True)
        return _

    lax.fori_loop(0, NCH, body, None)
    plsc.subcore_barrier()
    pltpu.sync_copy(acc.at[pl.ds(base, ROWS_PER_TILE)],
                    out_hbm.at[cid, pl.ds(base, ROWS_PER_TILE)])


@functools.cache
def _deg_call():
    return pl.kernel(
        _deg_body,
        out_type=jax.ShapeDtypeStruct((NC, NPAD, D), jnp.float32),
        mesh=_mesh(),
        scratch_types=[
            pltpu.VMEM((NCH, CHUNK), jnp.int32),
            pltpu.VMEM((CHUNK, D), jnp.float32),
            pltpu.SemaphoreType.DMA,
            pltpu.VMEM_SHARED((NPAD, D), jnp.float32),
        ],
    )


def _mp_body(xs_hbm, srcp_hbm, dstp_hbm, out_hbm, src_v, dst_v, buf0, buf1,
             g0, g1, s0, s1, acc):
    cid = lax.axis_index("c")
    sid = lax.axis_index("s")
    wid = sid * NC + cid
    pltpu.sync_copy(srcp_hbm.at[wid], src_v)
    pltpu.sync_copy(dstp_hbm.at[wid], dst_v)
    _zero_fill(buf0, MCH, D, 0.0)
    base = sid * ROWS_PER_TILE
    for i in range(ROWS_PER_TILE // MCH):
        pltpu.sync_copy(buf0, acc.at[pl.ds(base + i * MCH, MCH)])
    plsc.subcore_barrier()

    def body(p, _):
        sa = src_v.at[p, pl.ds(0, MCH)]
        sb = src_v.at[p, pl.ds(MCH, MCH)]
        da = dst_v.at[p, pl.ds(0, MCH)]
        db = dst_v.at[p, pl.ds(MCH, MCH)]
        pltpu.async_copy(xs_hbm.at[sa], buf0, g0)
        pltpu.async_copy(xs_hbm.at[sb], buf1, g1)
        pltpu.make_async_copy(xs_hbm.at[sa], buf0, g0).wait()
        pltpu.async_copy(buf0, acc.at[da], s0, add=True)
        pltpu.make_async_copy(xs_hbm.at[sb], buf1, g1).wait()
        pltpu.async_copy(buf1, acc.at[db], s1, add=True)
        pltpu.make_async_copy(buf0, acc.at[da], s0).wait()
        pltpu.make_async_copy(buf1, acc.at[db], s1).wait()
        return _

    lax.fori_loop(0, NCH, body, None)
    plsc.subcore_barrier()
    pltpu.sync_copy(acc.at[pl.ds(base, ROWS_PER_TILE)],
                    out_hbm.at[cid, pl.ds(base, ROWS_PER_TILE)])


@functools.cache
def _mp_call():
    return pl.kernel(
        _mp_body,
        out_type=jax.ShapeDtypeStruct((NC, NPAD, D), jnp.float32),
        mesh=_mesh(),
        scratch_types=[
            pltpu.VMEM((NCH, CHUNK), jnp.int32),
            pltpu.VMEM((NCH, CHUNK), jnp.int32),
            pltpu.VMEM((MCH, D), jnp.float32),
            pltpu.VMEM((MCH, D), jnp.float32),
            pltpu.SemaphoreType.DMA,
            pltpu.SemaphoreType.DMA,
            pltpu.SemaphoreType.DMA,
            pltpu.SemaphoreType.DMA,
            pltpu.VMEM_SHARED((NPAD, D), jnp.float32),
        ],
    )


_BR = 1024
_GRID = NPAD // _BR


def _dinv_of(degw_ref):
    deg = degw_ref[0, :, 0] + degw_ref[1, :, 0] + 1.0
    return 1.0 / jnp.sqrt(deg)


def _k1_body(x_ref, w_ref, degw_ref, o_ref):
    dinv = _dinv_of(degw_ref)
    xw = jnp.dot(x_ref[...], w_ref[...], preferred_element_type=jnp.float32)
    o_ref[...] = xw * dinv[:, None]


def _k23_body(p_ref, xs_ref, degw_ref, b_ref, w_ref, o_ref):
    dinv = _dinv_of(degw_ref)
    t = (p_ref[0] + p_ref[1] + xs_ref[...]) * dinv[:, None] + b_ref[...]
    h = jnp.maximum(t, 0.0)
    o_ref[...] = jnp.dot(h, w_ref[...], preferred_element_type=jnp.float32) * dinv[:, None]


def _k4_body(p_ref, xs_ref, degw_ref, b_ref, o_ref):
    dinv = _dinv_of(degw_ref)
    t = (p_ref[0] + p_ref[1] + xs_ref[...]) * dinv[:, None] + b_ref[...]
    m = jnp.max(t, axis=1, keepdims=True)
    lse = jnp.log(jnp.sum(jnp.exp(t - m), axis=1, keepdims=True)) + m
    o_ref[...] = t - lse


_deg_spec = pl.BlockSpec((NC, _BR, D), lambda i: (0, i, 0))
_p_spec = pl.BlockSpec((NC, _BR, D), lambda i: (0, i, 0))
_row_spec = pl.BlockSpec((_BR, D), lambda i: (i, 0))
_w_spec = pl.BlockSpec((D, D), lambda i: (0, 0))
_b_spec = pl.BlockSpec((1, D), lambda i: (0, 0))
_out_shape = jax.ShapeDtypeStruct((NPAD, D), jnp.float32)

_k1_call = pl.pallas_call(
    _k1_body, grid=(_GRID,),
    in_specs=[_row_spec, _w_spec, _deg_spec],
    out_specs=_row_spec, out_shape=_out_shape)

_k23_call = pl.pallas_call(
    _k23_body, grid=(_GRID,),
    in_specs=[_p_spec, _row_spec, _deg_spec, _b_spec, _w_spec],
    out_specs=_row_spec, out_shape=_out_shape)

_k4_call = pl.pallas_call(
    _k4_body, grid=(_GRID,),
    in_specs=[_p_spec, _row_spec, _deg_spec, _b_spec],
    out_specs=_row_spec, out_shape=_out_shape)


def _pack_edges(idx):
    w = idx.reshape(NW, EPW)
    w = jnp.pad(w, ((0, 0), (0, EPW_PAD - EPW)), constant_values=N)
    return w.reshape(NW, NCH, CHUNK)


def kernel(node_feature, edge_index, batch, W1, b1, W2, b2, W3, b3):
    x = jnp.pad(node_feature, ((0, NPAD - N), (0, 0)))
    srcp = _pack_edges(edge_index[0])
    dstp = _pack_edges(edge_index[1])

    degw = _deg_call()(dstp)
    xs1 = _k1_call(x, W1, degw)
    p1 = _mp_call()(xs1, srcp, dstp)
    xs2 = _k23_call(p1, xs1, degw, b1.reshape(1, D), W2)
    p2 = _mp_call()(xs2, srcp, dstp)
    xs3 = _k23_call(p2, xs2, degw, b2.reshape(1, D), W3)
    p3 = _mp_call()(xs3, srcp, dstp)
    out = _k4_call(p3, xs3, degw, b3.reshape(1, D))
    return out[:N]


# speedup vs baseline: 1.1316x; 1.1316x over previous
import functools

import jax
import jax.numpy as jnp
from jax import lax
from jax.experimental import pallas as pl
from jax.experimental.pallas import tpu as pltpu
from jax.experimental.pallas import tpu_sc as plsc

N = 10000
NPAD = 10240
E = 320000
D = 128
NC = 2
NS = 16
NW = NC * NS
EPW = E // NW
CHUNK = 128
NCH = 80
EPW_PAD = NCH * CHUNK
ROWS_PER_TILE = NPAD // NS

@functools.cache
def _mesh():
    return plsc.VectorSubcoreMesh(core_axis_name="c", subcore_axis_name="s",
                                  num_cores=NC, num_subcores=NS)


def _zero_fill(buf, rows, cols, val=0.0):
    v = jnp.full((16,), val, jnp.float32)

    def row(r, _):
        def col(c, __):
            buf[r, pl.ds(c * 16, 16)] = v
            return __
        return lax.fori_loop(0, cols // 16, col, _)

    lax.fori_loop(0, rows, row, None)


def _deg_body(dstp_hbm, out_hbm, dst_v, ones_v, sem, acc):
    cid = lax.axis_index("c")
    sid = lax.axis_index("s")
    wid = sid * NC + cid
    pltpu.sync_copy(dstp_hbm.at[wid], dst_v)
    _zero_fill(ones_v, CHUNK, D, 0.0)
    base = sid * ROWS_PER_TILE
    for i in range(ROWS_PER_TILE // CHUNK):
        pltpu.sync_copy(ones_v, acc.at[pl.ds(base + i * CHUNK, CHUNK)])
    _zero_fill(ones_v, CHUNK, D, 1.0)
    plsc.subcore_barrier()

    def body(j, _):
        pltpu.sync_copy(ones_v, acc.at[dst_v.at[j]], add=True)
        return _

    lax.fori_loop(0, NCH, body, None)
    plsc.subcore_barrier()
    pltpu.sync_copy(acc.at[pl.ds(base, ROWS_PER_TILE)],
                    out_hbm.at[cid, pl.ds(base, ROWS_PER_TILE)])


@functools.cache
def _deg_call():
    return pl.kernel(
        _deg_body,
        out_type=jax.ShapeDtypeStruct((NC, NPAD, D), jnp.float32),
        mesh=_mesh(),
        scratch_types=[
            pltpu.VMEM((NCH, CHUNK), jnp.int32),
            pltpu.VMEM((CHUNK, D), jnp.float32),
            pltpu.SemaphoreType.DMA,
            pltpu.VMEM_SHARED((NPAD, D), jnp.float32),
        ],
    )


def _mp_body(xs_hbm, srcp_hbm, dstp_hbm, out_hbm, src_v, dst_r, buf,
             gsem0, gsem1, i0, i1, acc):
    cid = lax.axis_index("c")
    sid = lax.axis_index("s")
    wid = sid * NC + cid
    pltpu.sync_copy(srcp_hbm.at[wid], src_v)
    pltpu.async_copy(dstp_hbm.at[wid, pl.ds(0, 1)], dst_r.at[pl.ds(0, 1)], i0)
    pltpu.async_copy(dstp_hbm.at[wid, pl.ds(1, 1)], dst_r.at[pl.ds(1, 1)], i1)
    _zero_fill(buf.at[0], CHUNK, D, 0.0)
    base = sid * ROWS_PER_TILE
    for i in range(ROWS_PER_TILE // CHUNK):
        pltpu.sync_copy(buf.at[0], acc.at[pl.ds(base + i * CHUNK, CHUNK)])
    plsc.subcore_barrier()
    pltpu.async_copy(xs_hbm.at[src_v.at[0]], buf.at[0], gsem0)

    def pair(p, _):
        for s in range(2):
            j = 2 * p + s
            gs = (gsem0, gsem1)[s]
            go = (gsem0, gsem1)[1 - s]
            is_ = (i0, i1)[s]
            @pl.when(j + 1 < NCH)
            def _():
                pltpu.async_copy(xs_hbm.at[src_v.at[j + 1]], buf.at[1 - s], go)
            pltpu.make_async_copy(xs_hbm.at[src_v.at[j]], buf.at[s], gs).wait()
            pltpu.make_async_copy(dstp_hbm.at[wid, pl.ds(0, 1)],
                                  dst_r.at[pl.ds(s, 1)], is_).wait()
            pltpu.sync_copy(buf.at[s], acc.at[dst_r.at[s]], add=True)
            @pl.when(j + 2 < NCH)
            def _():
                pltpu.async_copy(dstp_hbm.at[wid, pl.ds(j + 2, 1)],
                                 dst_r.at[pl.ds(s, 1)], is_)
        return _

    lax.fori_loop(0, NCH // 2, pair, None)
    plsc.subcore_barrier()
    pltpu.sync_copy(acc.at[pl.ds(base, ROWS_PER_TILE)],
                    out_hbm.at[cid, pl.ds(base, ROWS_PER_TILE)])


@functools.cache
def _mp_call():
    return pl.kernel(
        _mp_body,
        out_type=jax.ShapeDtypeStruct((NC, NPAD, D), jnp.float32),
        mesh=_mesh(),
        scratch_types=[
            pltpu.VMEM((NCH, CHUNK), jnp.int32),
            pltpu.VMEM((2, CHUNK), jnp.int32),
            pltpu.VMEM((2, CHUNK, D), jnp.float32),
            pltpu.SemaphoreType.DMA,
            pltpu.SemaphoreType.DMA,
            pltpu.SemaphoreType.DMA,
            pltpu.SemaphoreType.DMA,
            pltpu.VMEM_SHARED((NPAD, D), jnp.float32),
        ],
    )


_BR = 1024
_GRID = NPAD // _BR


def _dinv_of(degw_ref):
    deg = degw_ref[0, :, 0] + degw_ref[1, :, 0] + 1.0
    return 1.0 / jnp.sqrt(deg)


def _k1_body(x_ref, w_ref, degw_ref, o_ref):
    dinv = _dinv_of(degw_ref)
    xw = jnp.dot(x_ref[...], w_ref[...], preferred_element_type=jnp.float32)
    o_ref[...] = xw * dinv[:, None]


def _k23_body(p_ref, xs_ref, degw_ref, b_ref, w_ref, o_ref):
    dinv = _dinv_of(degw_ref)
    t = (p_ref[0] + p_ref[1] + xs_ref[...]) * dinv[:, None] + b_ref[...]
    h = jnp.maximum(t, 0.0)
    o_ref[...] = jnp.dot(h, w_ref[...], preferred_element_type=jnp.float32) * dinv[:, None]


def _k4_body(p_ref, xs_ref, degw_ref, b_ref, o_ref):
    dinv = _dinv_of(degw_ref)
    t = (p_ref[0] + p_ref[1] + xs_ref[...]) * dinv[:, None] + b_ref[...]
    m = jnp.max(t, axis=1, keepdims=True)
    lse = jnp.log(jnp.sum(jnp.exp(t - m), axis=1, keepdims=True)) + m
    o_ref[...] = t - lse


_deg_spec = pl.BlockSpec((NC, _BR, D), lambda i: (0, i, 0))
_p_spec = pl.BlockSpec((NC, _BR, D), lambda i: (0, i, 0))
_row_spec = pl.BlockSpec((_BR, D), lambda i: (i, 0))
_w_spec = pl.BlockSpec((D, D), lambda i: (0, 0))
_b_spec = pl.BlockSpec((1, D), lambda i: (0, 0))
_out_shape = jax.ShapeDtypeStruct((NPAD, D), jnp.float32)

_k1_call = pl.pallas_call(
    _k1_body, grid=(_GRID,),
    in_specs=[_row_spec, _w_spec, _deg_spec],
    out_specs=_row_spec, out_shape=_out_shape)

_k23_call = pl.pallas_call(
    _k23_body, grid=(_GRID,),
    in_specs=[_p_spec, _row_spec, _deg_spec, _b_spec, _w_spec],
    out_specs=_row_spec, out_shape=_out_shape)

_k4_call = pl.pallas_call(
    _k4_body, grid=(_GRID,),
    in_specs=[_p_spec, _row_spec, _deg_spec, _b_spec],
    out_specs=_row_spec, out_shape=_out_shape)


def _pack_edges(idx):
    w = idx.reshape(NW, EPW)
    w = jnp.pad(w, ((0, 0), (0, EPW_PAD - EPW)), constant_values=N)
    return w.reshape(NW, NCH, CHUNK)


def kernel(node_feature, edge_index, batch, W1, b1, W2, b2, W3, b3):
    x = jnp.pad(node_feature, ((0, NPAD - N), (0, 0)))
    srcp = _pack_edges(edge_index[0])
    dstp = _pack_edges(edge_index[1])

    degw = _deg_call()(dstp)
    xs1 = _k1_call(x, W1, degw)
    p1 = _mp_call()(xs1, srcp, dstp)
    xs2 = _k23_call(p1, xs1, degw, b1.reshape(1, D), W2)
    p2 = _mp_call()(xs2, srcp, dstp)
    xs3 = _k23_call(p2, xs2, degw, b2.reshape(1, D), W3)
    p3 = _mp_call()(xs3, srcp, dstp)
    out = _k4_call(p3, xs3, degw, b3.reshape(1, D))
    return out[:N]


# speedup vs baseline: 1.4205x; 1.2553x over previous
import functools

import jax
import jax.numpy as jnp
from jax import lax
from jax.experimental import pallas as pl
from jax.experimental.pallas import tpu as pltpu
from jax.experimental.pallas import tpu_sc as plsc

N = 10000
NPAD = 10240
E = 320000
D = 128
NC = 2
NS = 16
NW = NC * NS
EPW = E // NW
CHUNK = 128
NCH = (EPW + CHUNK - 1) // CHUNK
EPW_PAD = NCH * CHUNK
ROWS_PER_TILE = NPAD // NS

@functools.cache
def _mesh():
    return plsc.VectorSubcoreMesh(core_axis_name="c", subcore_axis_name="s",
                                  num_cores=NC, num_subcores=NS)


def _zero_fill(buf, rows, cols, val=0.0):
    v = jnp.full((16,), val, jnp.float32)

    def row(r, _):
        def col(c, __):
            buf[r, pl.ds(c * 16, 16)] = v
            return __
        return lax.fori_loop(0, cols // 16, col, _)

    lax.fori_loop(0, rows, row, None)


def _deg_body(dstp_hbm, out_hbm, dst_v, ones_v, sem, acc):
    cid = lax.axis_index("c")
    sid = lax.axis_index("s")
    wid = sid * NC + cid
    pltpu.sync_copy(dstp_hbm.at[wid], dst_v)
    _zero_fill(ones_v, CHUNK, D, 0.0)
    base = sid * ROWS_PER_TILE
    for i in range(ROWS_PER_TILE // CHUNK):
        pltpu.sync_copy(ones_v, acc.at[pl.ds(base + i * CHUNK, CHUNK)])
    _zero_fill(ones_v, CHUNK, D, 1.0)
    plsc.subcore_barrier()

    def body(j, _):
        pltpu.sync_copy(ones_v, acc.at[dst_v.at[j]], add=True)
        return _

    lax.fori_loop(0, NCH, body, None)
    plsc.subcore_barrier()
    pltpu.sync_copy(acc.at[pl.ds(base, ROWS_PER_TILE)],
                    out_hbm.at[cid, pl.ds(base, ROWS_PER_TILE)])


@functools.cache
def _deg_call():
    return pl.kernel(
        _deg_body,
        out_type=jax.ShapeDtypeStruct((NC, NPAD, D), jnp.float32),
        mesh=_mesh(),
        scratch_types=[
            pltpu.VMEM((NCH, CHUNK), jnp.int32),
            pltpu.VMEM((CHUNK, D), jnp.float32),
            pltpu.SemaphoreType.DMA,
            pltpu.VMEM_SHARED((NPAD, D), jnp.float32),
        ],
    )


def _mp_body(xs_hbm, srcp_hbm, dstp_hbm, out_hbm, src_v, dst_v, buf, gsem, acc):
    cid = lax.axis_index("c")
    sid = lax.axis_index("s")
    wid = sid * NC + cid
    pltpu.sync_copy(srcp_hbm.at[wid], src_v)
    pltpu.sync_copy(dstp_hbm.at[wid], dst_v)
    _zero_fill(buf, CHUNK, D, 0.0)
    base = sid * ROWS_PER_TILE
    for i in range(ROWS_PER_TILE // CHUNK):
        pltpu.sync_copy(buf, acc.at[pl.ds(base + i * CHUNK, CHUNK)])
    plsc.subcore_barrier()

    def body(j, _):
        pltpu.async_copy(xs_hbm.at[src_v.at[j]], buf, gsem).wait()
        pltpu.sync_copy(buf, acc.at[dst_v.at[j]], add=True)
        return _

    lax.fori_loop(0, NCH, body, None)
    plsc.subcore_barrier()
    pltpu.sync_copy(acc.at[pl.ds(base, ROWS_PER_TILE)],
                    out_hbm.at[cid, pl.ds(base, ROWS_PER_TILE)])


@functools.cache
def _mp_call():
    return pl.kernel(
        _mp_body,
        out_type=jax.ShapeDtypeStruct((NC, NPAD, D), jnp.float32),
        mesh=_mesh(),
        scratch_types=[
            pltpu.VMEM((NCH, CHUNK), jnp.int32),
            pltpu.VMEM((NCH, CHUNK), jnp.int32),
            pltpu.VMEM((CHUNK, D), jnp.float32),
            pltpu.SemaphoreType.DMA,
            pltpu.VMEM_SHARED((NPAD, D), jnp.float32),
        ],
    )


_BR = 1024
_GRID = NPAD // _BR


def _dinv_of(degw_ref):
    deg = degw_ref[0, :, 0] + degw_ref[1, :, 0] + 1.0
    return 1.0 / jnp.sqrt(deg)


def _k1_body(x_ref, w_ref, degw_ref, o_ref):
    dinv = _dinv_of(degw_ref)
    xw = jnp.dot(x_ref[...], w_ref[...], preferred_element_type=jnp.float32)
    o_ref[...] = xw * dinv[:, None]


def _k23_body(p_ref, xs_ref, degw_ref, b_ref, w_ref, o_ref):
    dinv = _dinv_of(degw_ref)
    t = (p_ref[0] + p_ref[1] + xs_ref[...]) * dinv[:, None] + b_ref[...]
    h = jnp.maximum(t, 0.0)
    o_ref[...] = jnp.dot(h, w_ref[...], preferred_element_type=jnp.float32) * dinv[:, None]


def _k4_body(p_ref, xs_ref, degw_ref, b_ref, o_ref):
    dinv = _dinv_of(degw_ref)
    t = (p_ref[0] + p_ref[1] + xs_ref[...]) * dinv[:, None] + b_ref[...]
    m = jnp.max(t, axis=1, keepdims=True)
    lse = jnp.log(jnp.sum(jnp.exp(t - m), axis=1, keepdims=True)) + m
    o_ref[...] = t - lse


_deg_spec = pl.BlockSpec((NC, _BR, D), lambda i: (0, i, 0))
_p_spec = pl.BlockSpec((NC, _BR, D), lambda i: (0, i, 0))
_row_spec = pl.BlockSpec((_BR, D), lambda i: (i, 0))
_w_spec = pl.BlockSpec((D, D), lambda i: (0, 0))
_b_spec = pl.BlockSpec((1, D), lambda i: (0, 0))
_out_shape = jax.ShapeDtypeStruct((NPAD, D), jnp.float32)

_k1_call = pl.pallas_call(
    _k1_body, grid=(_GRID,),
    in_specs=[_row_spec, _w_spec, _deg_spec],
    out_specs=_row_spec, out_shape=_out_shape)

_k23_call = pl.pallas_call(
    _k23_body, grid=(_GRID,),
    in_specs=[_p_spec, _row_spec, _deg_spec, _b_spec, _w_spec],
    out_specs=_row_spec, out_shape=_out_shape)

_k4_call = pl.pallas_call(
    _k4_body, grid=(_GRID,),
    in_specs=[_p_spec, _row_spec, _deg_spec, _b_spec],
    out_specs=_row_spec, out_shape=_out_shape)


def _pack_edges(idx):
    w = idx.reshape(NW, EPW)
    w = jnp.pad(w, ((0, 0), (0, EPW_PAD - EPW)), constant_values=N)
    return w.reshape(NW, NCH, CHUNK)


def kernel(node_feature, edge_index, batch, W1, b1, W2, b2, W3, b3):
    x = jnp.pad(node_feature, ((0, NPAD - N), (0, 0)))
    srcp = _pack_edges(edge_index[0])
    dstp = _pack_edges(edge_index[1])

    degw = _deg_call()(dstp)
    xs1 = _k1_call(x, W1, degw)
    p1 = _mp_call()(xs1, srcp, dstp)
    xs2 = _k23_call(p1, xs1, degw, b1.reshape(1, D), W2)
    p2 = _mp_call()(xs2, srcp, dstp)
    xs3 = _k23_call(p2, xs2, degw, b2.reshape(1, D), W3)
    p3 = _mp_call()(xs3, srcp, dstp)
    out = _k4_call(p3, xs3, degw, b3.reshape(1, D))
    return out[:N]
